# baseline (speedup 1.0000x reference)
import functools

import jax
import jax.numpy as jnp
from jax import lax
from jax.experimental import pallas as pl
from jax.experimental.pallas import tpu as pltpu, tpu_sc as plsc

_info = plsc.get_sparse_core_info()
_NC = _info.num_cores
_NS = _info.num_subcores
_NW = _NC * _NS

_CHUNK = 256


@functools.lru_cache(maxsize=None)
def _make_gather(ns: int, nb: int, D: int):
    assert D == 64
    b_per_w = nb // _NW
    hpw = b_per_w // _CHUNK
    n_chunks = ns * hpw

    mesh = plsc.VectorSubcoreMesh(core_axis_name="c", subcore_axis_name="s")

    @functools.partial(
        pl.kernel,
        out_type=jax.ShapeDtypeStruct((ns, D, nb), jnp.float32),
        mesh=mesh,
        scratch_types=(
            [pltpu.VMEM((_CHUNK,), jnp.int32) for _ in range(2)]
            + [pltpu.VMEM((_CHUNK, D), jnp.float32) for _ in range(2)]
            + [pltpu.VMEM((D, _CHUNK), jnp.float32) for _ in range(2)]
            + [pltpu.SemaphoreType.DMA for _ in range(4)]
        ),
        compiler_params=pltpu.CompilerParams(
            use_tc_tiling_on_sc=False, needs_layout_passes=False
        ),
    )
    def gather_kernel(table_hbm, idx_hbm, out_hbm, *refs):
        ih = refs[0:2]
        rows = refs[2:4]
        tb = refs[4:6]
        gsem = refs[6:8]
        osem = refs[8:10]

        wid = lax.axis_index("s") * _NC + lax.axis_index("c")
        bbase = wid * b_per_w

        iota = lax.iota(jnp.int32, 16)
        tj = [(iota + j) & 15 for j in range(16)]

        def split(c):
            s = c // hpw
            b0 = bbase + (c % hpw) * _CHUNK
            return s, b0

        def stage_idx(c, b):
            s, b0 = split(c)
            pltpu.sync_copy(idx_hbm.at[pl.ds(s * nb + b0, _CHUNK)], ih[b])

        def gather_copy(b):
            return pltpu.make_async_copy(table_hbm.at[ih[b]], rows[b], gsem[b])

        def out_copy(c, b):
            s, b0 = split(c)
            return pltpu.make_async_copy(
                tb[b], out_hbm.at[s, :, pl.ds(b0, _CHUNK)], osem[b]
            )

        stage_idx(0, 0)
        gather_copy(0).start()

        def step(c, b):
            @pl.when(c + 1 < n_chunks)
            def _():
                stage_idx(c + 1, 1 - b)
                gather_copy(1 - b).start()

            gather_copy(b).wait()

            @pl.when(c >= 2)
            def _():
                out_copy(c - 2, b).wait()

            def blk(R, carry):
                rr = R * 16 + iota
                for j in range(16):
                    for C in range(4):
                        cd = tj[j] + (16 * C)
                        vals = plsc.load_gather(rows[b], [rr, cd])
                        plsc.store_scatter(tb[b], [cd, rr], vals)
                return carry

            lax.fori_loop(0, _CHUNK // 16, blk, 0)
            out_copy(c, b).start()

        def pair(g, carry):
            step(2 * g, 0)
            step(2 * g + 1, 1)
            return carry

        lax.fori_loop(0, n_chunks // 2, pair, 0)

        out_copy(n_chunks - 2, 0).wait()
        out_copy(n_chunks - 1, 1).wait()

    return gather_kernel


def kernel(indices, table):
    nb, ns = indices.shape
    D = table.shape[1]
    flat = indices.T.reshape(nb * ns).astype(jnp.int32)
    out = _make_gather(ns, nb, D)(table, flat)
    return out.transpose(2, 0, 1)


# speedup vs baseline: 1.1889x; 1.1889x over previous
import functools

import jax
import jax.numpy as jnp
from jax import lax
from jax.experimental import pallas as pl
from jax.experimental.pallas import tpu as pltpu, tpu_sc as plsc

_info = plsc.get_sparse_core_info()
_NC = _info.num_cores
_NS = _info.num_subcores
_NW = _NC * _NS


@functools.lru_cache(maxsize=None)
def _make_gather(ns: int, nb: int, D: int):
    b_per_w = nb // _NW
    n_chunks = ns

    mesh = plsc.VectorSubcoreMesh(core_axis_name="c", subcore_axis_name="s")

    @functools.partial(
        pl.kernel,
        out_type=jax.ShapeDtypeStruct((ns, nb, D), jnp.float32),
        mesh=mesh,
        scratch_types=(
            [pltpu.VMEM((b_per_w,), jnp.int32) for _ in range(2)]
            + [pltpu.VMEM((b_per_w, D), jnp.float32) for _ in range(2)]
            + [pltpu.SemaphoreType.DMA for _ in range(4)]
        ),
        compiler_params=pltpu.CompilerParams(use_tc_tiling_on_sc=False),
    )
    def gather_kernel(table_hbm, idx_hbm, out_hbm, *refs):
        ih = refs[0:2]
        rows = refs[2:4]
        gsem = refs[4:6]
        osem = refs[6:8]

        wid = lax.axis_index("s") * _NC + lax.axis_index("c")
        b0 = wid * b_per_w

        def stage_idx(c, b):
            pltpu.sync_copy(idx_hbm.at[pl.ds(c * nb + b0, b_per_w)], ih[b])

        def gather_copy(b):
            return pltpu.make_async_copy(table_hbm.at[ih[b]], rows[b], gsem[b])

        def out_copy(c, b):
            return pltpu.make_async_copy(
                rows[b], out_hbm.at[c, pl.ds(b0, b_per_w), :], osem[b]
            )

        stage_idx(0, 0)
        gather_copy(0).start()

        def step(c, b):
            @pl.when(c + 1 < n_chunks)
            def _():
                stage_idx(c + 1, 1 - b)

                @pl.when(c >= 1)
                def _():
                    out_copy(c - 1, 1 - b).wait()

                gather_copy(1 - b).start()

            gather_copy(b).wait()
            out_copy(c, b).start()

        def pair(g, carry):
            step(2 * g, 0)
            step(2 * g + 1, 1)
            return carry

        lax.fori_loop(0, n_chunks // 2, pair, 0)

        out_copy(n_chunks - 2, 0).wait()
        out_copy(n_chunks - 1, 1).wait()

    return gather_kernel


def kernel(indices, table):
    nb, ns = indices.shape
    D = table.shape[1]
    flat = indices.T.reshape(nb * ns).astype(jnp.int32)
    out = _make_gather(ns, nb, D)(table, flat)
    return out.transpose(1, 0, 2)
